# Initial kernel scaffold; baseline (speedup 1.0000x reference)
import jax
import jax.numpy as jnp
from jax.experimental import pallas as pl


def kernel(x, edge_index, edge_weight, batch, gamma, beta, Ws, bs):
    raise NotImplementedError("write your pallas kernel here")


# speedup vs baseline: 45.9699x; 45.9699x over previous
import functools

import jax
import jax.numpy as jnp
from jax import lax
from jax.experimental import pallas as pl
from jax.experimental.pallas import tpu as pltpu
from jax.experimental.pallas import tpu_sc as plsc

_N = 10000
_E = 320000
_D = 128
_T = 10
_G = 128

_NC = 2
_NS = 16
_NW = _NC * _NS


_NR = 80


def _sc_col_degree(col, ew):
    ept = _E // _NW
    ch = 2000
    nch = ept // ch
    mesh = plsc.VectorSubcoreMesh(core_axis_name="c", subcore_axis_name="s")

    @functools.partial(
        pl.kernel, mesh=mesh,
        out_type=jax.ShapeDtypeStruct((_NW, _NR, _D), jnp.float32),
        compiler_params=pltpu.CompilerParams(needs_layout_passes=False),
        scratch_types=[
            pltpu.VMEM((_NR, _D), jnp.float32),
            pltpu.VMEM((ch,), jnp.int32),
            pltpu.VMEM((ch,), jnp.float32),
        ],
    )
    def k(col_hbm, ew_hbm, out_hbm, acc, idxb, ewb):
        cc = lax.axis_index("c")
        ss = lax.axis_index("s")
        wid = ss * _NC + cc

        def zero(g, carry):
            acc[g // 8, pl.ds((g % 8) * 16, 16)] = jnp.zeros((16,), jnp.float32)
            return carry
        lax.fori_loop(0, _NR * (_D // 16), zero, 0)

        base = wid * ept

        def chunk(kk, carry):
            off = base + kk * ch
            pltpu.sync_copy(col_hbm.at[pl.ds(off, ch)], idxb)
            pltpu.sync_copy(ew_hbm.at[pl.ds(off, ch)], ewb)

            def grp(j, c2):
                idx = idxb[pl.ds(j * 16, 16)]
                w = ewb[pl.ds(j * 16, 16)]
                hi = lax.shift_right_logical(idx, 7)
                lo = lax.bitwise_and(idx, 127)
                plsc.addupdate_scatter(acc, [hi, lo], w)
                return c2
            lax.fori_loop(0, ch // 16, grp, 0)
            return carry
        lax.fori_loop(0, nch, chunk, 0)
        pltpu.sync_copy(acc, out_hbm.at[wid])

    return k(col, ew)


def _sc_edge_agg(xs, row, col, ew):
    epc = _E // _NC
    ept = epc // _NS
    ch = 80
    nch = ept // ch
    rpt = 640
    zr = 40
    mesh = plsc.VectorSubcoreMesh(core_axis_name="c", subcore_axis_name="s")

    @functools.partial(
        pl.kernel, mesh=mesh,
        out_type=jax.ShapeDtypeStruct((_NC, _N, _D), jnp.float32),
        compiler_params=pltpu.CompilerParams(needs_layout_passes=False),
        scratch_types=[
            pltpu.VMEM_SHARED((_N, _D), jnp.float32),
            pltpu.VMEM((zr, _D), jnp.float32),
            pltpu.VMEM((ch,), jnp.int32),
            pltpu.VMEM((ch,), jnp.int32),
            pltpu.VMEM((ch,), jnp.float32),
            pltpu.VMEM((ch, _D), jnp.float32),
            pltpu.SemaphoreType.DMA,
        ],
    )
    def k(xs_hbm, row_hbm, col_hbm, ew_hbm, out_hbm,
          shacc, zbuf, rowb, colb, ewb, rows_v, sem):
        cc = lax.axis_index("c")
        ss = lax.axis_index("s")

        def zzb(g, carry):
            zbuf[g // 8, pl.ds((g % 8) * 16, 16)] = jnp.zeros((16,), jnp.float32)
            return carry
        lax.fori_loop(0, zr * (_D // 16), zzb, 0)

        r0 = ss * rpt
        nz = lax.select(ss == _NS - 1, (_N - 15 * rpt) // zr, rpt // zr)

        def zsh(i, carry):
            pltpu.sync_copy(zbuf, shacc.at[pl.ds(r0 + i * zr, zr)])
            return carry
        lax.fori_loop(0, nz, zsh, 0)
        plsc.subcore_barrier()

        base = cc * epc + ss * ept

        def chunk(kk, carry):
            off = base + kk * ch
            pltpu.sync_copy(row_hbm.at[pl.ds(off, ch)], rowb)
            pltpu.sync_copy(col_hbm.at[pl.ds(off, ch)], colb)
            pltpu.sync_copy(ew_hbm.at[pl.ds(off, ch)], ewb)
            pltpu.async_copy(xs_hbm.at[rowb], rows_v, sem).wait()

            def pe(j, c2):
                wj = plsc.load_gather(ewb, [jnp.full((16,), j, jnp.int32)])
                for v in range(_D // 16):
                    sl = pl.ds(v * 16, 16)
                    rows_v[j, sl] = rows_v[j, sl] * wj
                return c2
            lax.fori_loop(0, ch, pe, 0)
            pltpu.sync_copy(rows_v, shacc.at[colb], add=True)
            return carry
        lax.fori_loop(0, nch, chunk, 0)
        plsc.subcore_barrier()

        def wout(i, carry):
            pltpu.sync_copy(shacc.at[pl.ds(r0 + i * zr, zr)],
                            out_hbm.at[cc, pl.ds(r0 + i * zr, zr)])
            return carry
        lax.fori_loop(0, nz, wout, 0)

    return k(xs, row, col, ew)


_BN = 2000
_NB = _N // _BN

_F32 = jnp.float32
_DN0 = (((0,), (0,)), ((), ()))
_DN1 = (((1,), (1,)), ((), ()))


def _tc_stats(x):
    def body(x_ref, o_ref):
        i = pl.program_id(0)
        xv = x_ref[...]
        blk = jnp.concatenate(
            [jnp.sum(xv, axis=0, keepdims=True),
             jnp.sum(xv * xv, axis=0, keepdims=True)], axis=0)

        @pl.when(i == 0)
        def _():
            o_ref[...] = blk

        @pl.when(i > 0)
        def _():
            o_ref[...] += blk

    return pl.pallas_call(
        body,
        grid=(_NB,),
        in_specs=[pl.BlockSpec((_BN, _D), lambda i: (i, 0))],
        out_specs=pl.BlockSpec((2, _D), lambda i: (0, 0)),
        out_shape=jax.ShapeDtypeStruct((2, _D), _F32),
    )(x)


def _tc_prep(x, stats, gamma2, beta2, degp):
    def body(x_ref, st_ref, g_ref, b_ref, dp_ref, xb_ref, xs_ref, dis_ref):
        xv = x_ref[...]
        mean = st_ref[0:1, :] * (1.0 / _N)
        var = st_ref[1:2, :] * (1.0 / _N) - mean * mean
        xb = (xv - mean) * lax.rsqrt(var + 1e-5) * g_ref[...] + b_ref[...]
        deg = jnp.sum(dp_ref[...], axis=1, keepdims=True) + 1.0
        dis = lax.rsqrt(deg)
        xb_ref[...] = xb
        xs_ref[...] = xb * dis
        dis_ref[...] = dis

    return pl.pallas_call(
        body,
        grid=(_NB,),
        in_specs=[
            pl.BlockSpec((_BN, _D), lambda i: (i, 0)),
            pl.BlockSpec((2, _D), lambda i: (0, 0)),
            pl.BlockSpec((1, _D), lambda i: (0, 0)),
            pl.BlockSpec((1, _D), lambda i: (0, 0)),
            pl.BlockSpec((_BN, _NW), lambda i: (i, 0)),
        ],
        out_specs=[
            pl.BlockSpec((_BN, _D), lambda i: (i, 0)),
            pl.BlockSpec((_BN, _D), lambda i: (i, 0)),
            pl.BlockSpec((_BN, 1), lambda i: (i, 0)),
        ],
        out_shape=[
            jax.ShapeDtypeStruct((_N, _D), _F32),
            jax.ShapeDtypeStruct((_N, _D), _F32),
            jax.ShapeDtypeStruct((_N, 1), _F32),
        ],
    )(x, stats, gamma2, beta2, degp)


def _onehot(bat):
    return (bat == lax.broadcasted_iota(jnp.int32, (1, _G), 1)).astype(_F32)


def _squash_rows(s):
    n2 = jnp.sum(s * s, axis=1, keepdims=True)
    return (n2 / (1.0 + n2)) * s / jnp.sqrt(n2 + 1e-8)


def _tc_finalize(p0, p1p, xs, xb, dis, bat2, Ws, bs):
    def body(p0_ref, p1_ref, xs_ref, xb_ref, dis_ref, bat_ref, Ws_ref, bs_ref,
             agg_ref, SX_ref, cnt_ref, pout_ref, qout_ref, SA_sc):
        i = pl.program_id(0)
        agg = dis_ref[...] * (p0_ref[...] + p1_ref[...] + xs_ref[...])
        agg_ref[...] = agg
        M = _onehot(bat_ref[...])
        sa = lax.dot_general(M, agg, _DN0, preferred_element_type=_F32)
        sx = lax.dot_general(M, xb_ref[...], _DN0, preferred_element_type=_F32)
        cn = lax.dot_general(M, jnp.ones((_BN, 1), _F32), _DN0,
                             preferred_element_type=_F32)

        @pl.when(i == 0)
        def _():
            SA_sc[...] = sa
            SX_ref[...] = sx
            cnt_ref[...] = cn

        @pl.when(i > 0)
        def _():
            SA_sc[...] += sa
            SX_ref[...] += sx
            cnt_ref[...] += cn

        @pl.when(i == _NB - 1)
        def _():
            SA = SA_sc[...]
            cnt = cnt_ref[...]
            for t in range(_T):
                Wt = Ws_ref[t]
                bt = bs_ref[t:t + 1, :]
                s1 = (jnp.dot(SA, Wt, preferred_element_type=_F32)
                      + cnt * bt) * (1.0 / _T)
                v1 = _squash_rows(s1)
                pout_ref[t] = lax.dot_general(v1, Wt, _DN1,
                                              preferred_element_type=_F32)
                qout_ref[:, t:t + 1] = lax.dot_general(
                    v1, bt, _DN1, preferred_element_type=_F32)

    return pl.pallas_call(
        body,
        grid=(_NB,),
        in_specs=[
            pl.BlockSpec((_BN, _D), lambda i: (i, 0)),
            pl.BlockSpec((_BN, _D), lambda i: (i, 0)),
            pl.BlockSpec((_BN, _D), lambda i: (i, 0)),
            pl.BlockSpec((_BN, _D), lambda i: (i, 0)),
            pl.BlockSpec((_BN, 1), lambda i: (i, 0)),
            pl.BlockSpec((_BN, 1), lambda i: (i, 0)),
            pl.BlockSpec((_T, _D, _D), lambda i: (0, 0, 0)),
            pl.BlockSpec((_T, _D), lambda i: (0, 0)),
        ],
        out_specs=[
            pl.BlockSpec((_BN, _D), lambda i: (i, 0)),
            pl.BlockSpec((_G, _D), lambda i: (0, 0)),
            pl.BlockSpec((_G, 1), lambda i: (0, 0)),
            pl.BlockSpec((_T, _G, _D), lambda i: (0, 0, 0)),
            pl.BlockSpec((_G, _T), lambda i: (0, 0)),
        ],
        out_shape=[
            jax.ShapeDtypeStruct((_N, _D), _F32),
            jax.ShapeDtypeStruct((_G, _D), _F32),
            jax.ShapeDtypeStruct((_G, 1), _F32),
            jax.ShapeDtypeStruct((_T, _G, _D), _F32),
            jax.ShapeDtypeStruct((_G, _T), _F32),
        ],
        scratch_shapes=[pltpu.VMEM((_G, _D), _F32)],
    )(p0, p1p, xs, xb, dis, bat2, Ws, bs)


def _tc_node_pass(agg, bat2, p, q, Ws, bs, SX, cnt, final):
    def body(agg_ref, bat_ref, p_ref, q_ref, Ws_ref, bs_ref, SX_ref, cnt_ref,
             *rest):
        if final:
            (out_ref, A_sc, m_sc) = rest
        else:
            (pout_ref, qout_ref, A_sc, m_sc) = rest
        i = pl.program_id(0)
        agg = agg_ref[...]
        M = _onehot(bat_ref[...])
        qn = jnp.dot(M, q_ref[...], preferred_element_type=_F32)
        wcols = []
        for t in range(_T):
            pn = jnp.dot(M, p_ref[t], preferred_element_type=_F32)
            wcols.append(jnp.sum(agg * pn, axis=1, keepdims=True))
        logits = jnp.concatenate(wcols, axis=1) + qn
        mx = jnp.max(logits, axis=1, keepdims=True)
        e = jnp.exp(logits - mx)
        c = e / jnp.sum(e, axis=1, keepdims=True)

        @pl.when(i == 0)
        def _():
            A_sc[...] = jnp.zeros_like(A_sc)
            m_sc[...] = jnp.zeros_like(m_sc)

        m_sc[...] += lax.dot_general(M, c, _DN0, preferred_element_type=_F32)
        for t in range(_T):
            A_sc[t] += lax.dot_general(M, c[:, t:t + 1] * agg, _DN0,
                                       preferred_element_type=_F32)

        @pl.when(i == _NB - 1)
        def _():
            for t in range(_T):
                Wt = Ws_ref[t]
                bt = bs_ref[t:t + 1, :]
                s = (jnp.dot(A_sc[t], Wt, preferred_element_type=_F32)
                     + m_sc[:, t:t + 1] * bt)
                if final:
                    s = s + SX_ref[...] / jnp.maximum(cnt_ref[...], 1.0)
                v = _squash_rows(s)
                if final:
                    out_ref[:, t:t + 1] = jnp.sqrt(
                        jnp.sum(v * v, axis=1, keepdims=True))
                else:
                    pout_ref[t] = p_ref[t] + lax.dot_general(
                        v, Wt, _DN1, preferred_element_type=_F32)
                    qout_ref[:, t:t + 1] = q_ref[:, t:t + 1] + lax.dot_general(
                        v, bt, _DN1, preferred_element_type=_F32)

    if final:
        out_specs = [pl.BlockSpec((_G, _T), lambda i: (0, 0))]
        out_shape = [jax.ShapeDtypeStruct((_G, _T), _F32)]
    else:
        out_specs = [
            pl.BlockSpec((_T, _G, _D), lambda i: (0, 0, 0)),
            pl.BlockSpec((_G, _T), lambda i: (0, 0)),
        ]
        out_shape = [
            jax.ShapeDtypeStruct((_T, _G, _D), _F32),
            jax.ShapeDtypeStruct((_G, _T), _F32),
        ]

    res = pl.pallas_call(
        body,
        grid=(_NB,),
        in_specs=[
            pl.BlockSpec((_BN, _D), lambda i: (i, 0)),
            pl.BlockSpec((_BN, 1), lambda i: (i, 0)),
            pl.BlockSpec((_T, _G, _D), lambda i: (0, 0, 0)),
            pl.BlockSpec((_G, _T), lambda i: (0, 0)),
            pl.BlockSpec((_T, _D, _D), lambda i: (0, 0, 0)),
            pl.BlockSpec((_T, _D), lambda i: (0, 0)),
            pl.BlockSpec((_G, _D), lambda i: (0, 0)),
            pl.BlockSpec((_G, 1), lambda i: (0, 0)),
        ],
        out_specs=out_specs,
        out_shape=out_shape,
        scratch_shapes=[pltpu.VMEM((_T, _G, _D), _F32),
                        pltpu.VMEM((_G, _T), _F32)],
    )(agg, bat2, p, q, Ws, bs, SX, cnt)
    return res


def kernel(x, edge_index, edge_weight, batch, gamma, beta, Ws, bs):
    row = edge_index[0]
    col = edge_index[1]
    gamma2 = gamma.reshape(1, _D)
    beta2 = beta.reshape(1, _D)
    bat2 = batch.reshape(_N, 1)

    degp = _sc_col_degree(col, edge_weight)
    degp_t = degp.reshape(_NW, _NR * _D)[:, :_N].T
    stats = _tc_stats(x)
    xb, xs, dis = _tc_prep(x, stats, gamma2, beta2, degp_t)
    P = _sc_edge_agg(xs, row, col, edge_weight)
    agg, SX, cnt, p1, q1 = _tc_finalize(P[0], P[1], xs, xb, dis, bat2, Ws, bs)
    p12, q12 = _tc_node_pass(agg, bat2, p1, q1, Ws, bs, SX, cnt, final=False)
    (out,) = _tc_node_pass(agg, bat2, p12, q12, Ws, bs, SX, cnt, final=True)
    return out


# speedup vs baseline: 95.7889x; 2.0837x over previous
import functools

import jax
import jax.numpy as jnp
from jax import lax
from jax.experimental import pallas as pl
from jax.experimental.pallas import tpu as pltpu
from jax.experimental.pallas import tpu_sc as plsc

_N = 10000
_E = 320000
_D = 128
_T = 10
_G = 128

_NC = 2
_NS = 16
_NW = _NC * _NS


_NR = 80


def _sc_col_degree(col, ew):
    ept = _E // _NW
    ch = 2000
    nch = ept // ch
    mesh = plsc.VectorSubcoreMesh(core_axis_name="c", subcore_axis_name="s")

    @functools.partial(
        pl.kernel, mesh=mesh,
        out_type=jax.ShapeDtypeStruct((_NW, _NR, _D), jnp.float32),
        compiler_params=pltpu.CompilerParams(needs_layout_passes=False),
        scratch_types=[
            pltpu.VMEM((_NR, _D), jnp.float32),
            pltpu.VMEM((ch,), jnp.int32),
            pltpu.VMEM((ch,), jnp.float32),
        ],
    )
    def k(col_hbm, ew_hbm, out_hbm, acc, idxb, ewb):
        cc = lax.axis_index("c")
        ss = lax.axis_index("s")
        wid = ss * _NC + cc

        def zero(g, carry):
            acc[g // 8, pl.ds((g % 8) * 16, 16)] = jnp.zeros((16,), jnp.float32)
            return carry
        lax.fori_loop(0, _NR * (_D // 16), zero, 0)

        base = wid * ept

        def chunk(kk, carry):
            off = base + kk * ch
            pltpu.sync_copy(col_hbm.at[pl.ds(off, ch)], idxb)
            pltpu.sync_copy(ew_hbm.at[pl.ds(off, ch)], ewb)

            def grp(j, c2):
                idx = idxb[pl.ds(j * 16, 16)]
                w = ewb[pl.ds(j * 16, 16)]
                hi = lax.shift_right_logical(idx, 7)
                lo = lax.bitwise_and(idx, 127)
                plsc.addupdate_scatter(acc, [hi, lo], w)
                return c2
            lax.fori_loop(0, ch // 16, grp, 0)
            return carry
        lax.fori_loop(0, nch, chunk, 0)
        pltpu.sync_copy(acc, out_hbm.at[wid])

    return k(col, ew)


def _sc_edge_agg(xs, row, col, ew):
    epc = _E // _NC
    ept = epc // _NS
    ch = 40
    nch = ept // ch
    nbuf = 5
    rpt = 640
    mesh = plsc.VectorSubcoreMesh(core_axis_name="c", subcore_axis_name="s")

    scratch = [pltpu.VMEM_SHARED((_N, _D), jnp.float32)]
    for _ in range(nbuf):
        scratch += [
            pltpu.VMEM((ch,), jnp.int32),
            pltpu.VMEM((ch,), jnp.int32),
            pltpu.VMEM((ch,), jnp.float32),
            pltpu.VMEM((ch, _D), jnp.float32),
            pltpu.SemaphoreType.DMA,
            pltpu.SemaphoreType.DMA,
            pltpu.SemaphoreType.DMA,
        ]

    @functools.partial(
        pl.kernel, mesh=mesh,
        out_type=jax.ShapeDtypeStruct((_NC, _N, _D), jnp.float32),
        compiler_params=pltpu.CompilerParams(needs_layout_passes=False),
        scratch_types=scratch,
    )
    def k(xs_hbm, row_hbm, col_hbm, ew_hbm, out_hbm, shacc, *bufs):
        rowb = [bufs[7 * b + 0] for b in range(nbuf)]
        colb = [bufs[7 * b + 1] for b in range(nbuf)]
        ewb = [bufs[7 * b + 2] for b in range(nbuf)]
        rows = [bufs[7 * b + 3] for b in range(nbuf)]
        isem = [bufs[7 * b + 4] for b in range(nbuf)]
        gsem = [bufs[7 * b + 5] for b in range(nbuf)]
        ssem = [bufs[7 * b + 6] for b in range(nbuf)]
        cc = lax.axis_index("c")
        ss = lax.axis_index("s")

        def zzb(g, carry):
            rows[0][g // 8, pl.ds((g % 8) * 16, 16)] = (
                jnp.zeros((16,), jnp.float32))
            return carry
        lax.fori_loop(0, ch * (_D // 16), zzb, 0)

        r0 = ss * rpt
        nz = lax.select(ss == _NS - 1, (_N - 15 * rpt) // ch, rpt // ch)

        def zsh(i, carry):
            pltpu.sync_copy(rows[0], shacc.at[pl.ds(r0 + i * ch, ch)])
            return carry
        lax.fori_loop(0, nz, zsh, 0)
        plsc.subcore_barrier()

        base = cc * epc + ss * ept

        def fetch_idx(cp, bb):
            off = base + cp * ch
            pltpu.async_copy(row_hbm.at[pl.ds(off, ch)], rowb[bb], isem[bb])
            pltpu.async_copy(col_hbm.at[pl.ds(off, ch)], colb[bb], isem[bb])
            pltpu.async_copy(ew_hbm.at[pl.ds(off, ch)], ewb[bb], isem[bb])

        def wait_idx(cp, bb):
            off = base + cp * ch
            pltpu.make_async_copy(
                row_hbm.at[pl.ds(off, ch)], rowb[bb], isem[bb]).wait()
            pltpu.make_async_copy(
                col_hbm.at[pl.ds(off, ch)], colb[bb], isem[bb]).wait()
            pltpu.make_async_copy(
                ew_hbm.at[pl.ds(off, ch)], ewb[bb], isem[bb]).wait()

        for b in range(3):
            fetch_idx(b, b)
        for b in range(2):
            wait_idx(b, b)
            pltpu.async_copy(xs_hbm.at[rowb[b]], rows[b], gsem[b])

        def outer(kk, carry):
            for b in range(nbuf):
                c = kk * nbuf + b
                b3 = (b + 3) % nbuf
                b2 = (b + 2) % nbuf

                @pl.when(c + 3 < nch)
                def _(b3=b3, c=c):
                    @pl.when(c + 3 >= nbuf)
                    def _():
                        pltpu.make_async_copy(
                            rows[b3], shacc.at[colb[b3]], ssem[b3]).wait()
                    fetch_idx(c + 3, b3)

                @pl.when(c + 2 < nch)
                def _(b2=b2, c=c):
                    wait_idx(c + 2, b2)
                    pltpu.async_copy(xs_hbm.at[rowb[b2]], rows[b2], gsem[b2])

                pltpu.make_async_copy(xs_hbm.at[rowb[b]], rows[b],
                                      gsem[b]).wait()

                def sc8(i8, c2, b=b):
                    for u in range(8):
                        j = i8 * 8 + u
                        wj = plsc.load_gather(
                            ewb[b], [jnp.full((16,), j, jnp.int32)])
                        for v in range(_D // 16):
                            sl = pl.ds(v * 16, 16)
                            rows[b][j, sl] = rows[b][j, sl] * wj
                    return c2
                lax.fori_loop(0, ch // 8, sc8, 0)
                pltpu.async_copy(rows[b], shacc.at[colb[b]], ssem[b],
                                 add=True)
            return carry
        lax.fori_loop(0, nch // nbuf, outer, 0)
        for b in range(nbuf):
            pltpu.make_async_copy(rows[b], shacc.at[colb[b]], ssem[b]).wait()
        plsc.subcore_barrier()

        def wout(i, carry):
            pltpu.sync_copy(shacc.at[pl.ds(r0 + i * ch, ch)],
                            out_hbm.at[cc, pl.ds(r0 + i * ch, ch)])
            return carry
        lax.fori_loop(0, nz, wout, 0)

    return k(xs, row, col, ew)


_BN = 2000
_NB = _N // _BN

_F32 = jnp.float32
_DN0 = (((0,), (0,)), ((), ()))
_DN1 = (((1,), (1,)), ((), ()))


def _tc_stats(x):
    def body(x_ref, o_ref):
        i = pl.program_id(0)
        xv = x_ref[...]
        blk = jnp.concatenate(
            [jnp.sum(xv, axis=0, keepdims=True),
             jnp.sum(xv * xv, axis=0, keepdims=True)], axis=0)

        @pl.when(i == 0)
        def _():
            o_ref[...] = blk

        @pl.when(i > 0)
        def _():
            o_ref[...] += blk

    return pl.pallas_call(
        body,
        grid=(_NB,),
        in_specs=[pl.BlockSpec((_BN, _D), lambda i: (i, 0))],
        out_specs=pl.BlockSpec((2, _D), lambda i: (0, 0)),
        out_shape=jax.ShapeDtypeStruct((2, _D), _F32),
    )(x)


def _tc_prep(x, stats, gamma2, beta2, degp):
    def body(x_ref, st_ref, g_ref, b_ref, dp_ref, xb_ref, xs_ref, dis_ref):
        xv = x_ref[...]
        mean = st_ref[0:1, :] * (1.0 / _N)
        var = st_ref[1:2, :] * (1.0 / _N) - mean * mean
        xb = (xv - mean) * lax.rsqrt(var + 1e-5) * g_ref[...] + b_ref[...]
        deg = jnp.sum(dp_ref[...], axis=1, keepdims=True) + 1.0
        dis = lax.rsqrt(deg)
        xb_ref[...] = xb
        xs_ref[...] = xb * dis
        dis_ref[...] = dis

    return pl.pallas_call(
        body,
        grid=(_NB,),
        in_specs=[
            pl.BlockSpec((_BN, _D), lambda i: (i, 0)),
            pl.BlockSpec((2, _D), lambda i: (0, 0)),
            pl.BlockSpec((1, _D), lambda i: (0, 0)),
            pl.BlockSpec((1, _D), lambda i: (0, 0)),
            pl.BlockSpec((_BN, _NW), lambda i: (i, 0)),
        ],
        out_specs=[
            pl.BlockSpec((_BN, _D), lambda i: (i, 0)),
            pl.BlockSpec((_BN, _D), lambda i: (i, 0)),
            pl.BlockSpec((_BN, 1), lambda i: (i, 0)),
        ],
        out_shape=[
            jax.ShapeDtypeStruct((_N, _D), _F32),
            jax.ShapeDtypeStruct((_N, _D), _F32),
            jax.ShapeDtypeStruct((_N, 1), _F32),
        ],
    )(x, stats, gamma2, beta2, degp)


def _onehot(bat):
    return (bat == lax.broadcasted_iota(jnp.int32, (1, _G), 1)).astype(_F32)


def _squash_rows(s):
    n2 = jnp.sum(s * s, axis=1, keepdims=True)
    return (n2 / (1.0 + n2)) * s / jnp.sqrt(n2 + 1e-8)


def _tc_finalize(p0, p1p, xs, xb, dis, bat2, Ws, bs):
    def body(p0_ref, p1_ref, xs_ref, xb_ref, dis_ref, bat_ref, Ws_ref, bs_ref,
             agg_ref, SX_ref, cnt_ref, pout_ref, qout_ref, SA_sc):
        i = pl.program_id(0)
        agg = dis_ref[...] * (p0_ref[...] + p1_ref[...] + xs_ref[...])
        agg_ref[...] = agg
        M = _onehot(bat_ref[...])
        sa = lax.dot_general(M, agg, _DN0, preferred_element_type=_F32)
        sx = lax.dot_general(M, xb_ref[...], _DN0, preferred_element_type=_F32)
        cn = lax.dot_general(M, jnp.ones((_BN, 1), _F32), _DN0,
                             preferred_element_type=_F32)

        @pl.when(i == 0)
        def _():
            SA_sc[...] = sa
            SX_ref[...] = sx
            cnt_ref[...] = cn

        @pl.when(i > 0)
        def _():
            SA_sc[...] += sa
            SX_ref[...] += sx
            cnt_ref[...] += cn

        @pl.when(i == _NB - 1)
        def _():
            SA = SA_sc[...]
            cnt = cnt_ref[...]
            for t in range(_T):
                Wt = Ws_ref[t]
                bt = bs_ref[t:t + 1, :]
                s1 = (jnp.dot(SA, Wt, preferred_element_type=_F32)
                      + cnt * bt) * (1.0 / _T)
                v1 = _squash_rows(s1)
                pout_ref[t] = lax.dot_general(v1, Wt, _DN1,
                                              preferred_element_type=_F32)
                qout_ref[:, t:t + 1] = lax.dot_general(
                    v1, bt, _DN1, preferred_element_type=_F32)

    return pl.pallas_call(
        body,
        grid=(_NB,),
        in_specs=[
            pl.BlockSpec((_BN, _D), lambda i: (i, 0)),
            pl.BlockSpec((_BN, _D), lambda i: (i, 0)),
            pl.BlockSpec((_BN, _D), lambda i: (i, 0)),
            pl.BlockSpec((_BN, _D), lambda i: (i, 0)),
            pl.BlockSpec((_BN, 1), lambda i: (i, 0)),
            pl.BlockSpec((_BN, 1), lambda i: (i, 0)),
            pl.BlockSpec((_T, _D, _D), lambda i: (0, 0, 0)),
            pl.BlockSpec((_T, _D), lambda i: (0, 0)),
        ],
        out_specs=[
            pl.BlockSpec((_BN, _D), lambda i: (i, 0)),
            pl.BlockSpec((_G, _D), lambda i: (0, 0)),
            pl.BlockSpec((_G, 1), lambda i: (0, 0)),
            pl.BlockSpec((_T, _G, _D), lambda i: (0, 0, 0)),
            pl.BlockSpec((_G, _T), lambda i: (0, 0)),
        ],
        out_shape=[
            jax.ShapeDtypeStruct((_N, _D), _F32),
            jax.ShapeDtypeStruct((_G, _D), _F32),
            jax.ShapeDtypeStruct((_G, 1), _F32),
            jax.ShapeDtypeStruct((_T, _G, _D), _F32),
            jax.ShapeDtypeStruct((_G, _T), _F32),
        ],
        scratch_shapes=[pltpu.VMEM((_G, _D), _F32)],
    )(p0, p1p, xs, xb, dis, bat2, Ws, bs)


def _tc_node_pass(agg, bat2, p, q, Ws, bs, SX, cnt, final):
    def body(agg_ref, bat_ref, p_ref, q_ref, Ws_ref, bs_ref, SX_ref, cnt_ref,
             *rest):
        if final:
            (out_ref, A_sc, m_sc) = rest
        else:
            (pout_ref, qout_ref, A_sc, m_sc) = rest
        i = pl.program_id(0)
        agg = agg_ref[...]
        M = _onehot(bat_ref[...])
        qn = jnp.dot(M, q_ref[...], preferred_element_type=_F32)
        wcols = []
        for t in range(_T):
            pn = jnp.dot(M, p_ref[t], preferred_element_type=_F32)
            wcols.append(jnp.sum(agg * pn, axis=1, keepdims=True))
        logits = jnp.concatenate(wcols, axis=1) + qn
        mx = jnp.max(logits, axis=1, keepdims=True)
        e = jnp.exp(logits - mx)
        c = e / jnp.sum(e, axis=1, keepdims=True)

        @pl.when(i == 0)
        def _():
            A_sc[...] = jnp.zeros_like(A_sc)
            m_sc[...] = jnp.zeros_like(m_sc)

        m_sc[...] += lax.dot_general(M, c, _DN0, preferred_element_type=_F32)
        for t in range(_T):
            A_sc[t] += lax.dot_general(M, c[:, t:t + 1] * agg, _DN0,
                                       preferred_element_type=_F32)

        @pl.when(i == _NB - 1)
        def _():
            for t in range(_T):
                Wt = Ws_ref[t]
                bt = bs_ref[t:t + 1, :]
                s = (jnp.dot(A_sc[t], Wt, preferred_element_type=_F32)
                     + m_sc[:, t:t + 1] * bt)
                if final:
                    s = s + SX_ref[...] / jnp.maximum(cnt_ref[...], 1.0)
                v = _squash_rows(s)
                if final:
                    out_ref[:, t:t + 1] = jnp.sqrt(
                        jnp.sum(v * v, axis=1, keepdims=True))
                else:
                    pout_ref[t] = p_ref[t] + lax.dot_general(
                        v, Wt, _DN1, preferred_element_type=_F32)
                    qout_ref[:, t:t + 1] = q_ref[:, t:t + 1] + lax.dot_general(
                        v, bt, _DN1, preferred_element_type=_F32)

    if final:
        out_specs = [pl.BlockSpec((_G, _T), lambda i: (0, 0))]
        out_shape = [jax.ShapeDtypeStruct((_G, _T), _F32)]
    else:
        out_specs = [
            pl.BlockSpec((_T, _G, _D), lambda i: (0, 0, 0)),
            pl.BlockSpec((_G, _T), lambda i: (0, 0)),
        ]
        out_shape = [
            jax.ShapeDtypeStruct((_T, _G, _D), _F32),
            jax.ShapeDtypeStruct((_G, _T), _F32),
        ]

    res = pl.pallas_call(
        body,
        grid=(_NB,),
        in_specs=[
            pl.BlockSpec((_BN, _D), lambda i: (i, 0)),
            pl.BlockSpec((_BN, 1), lambda i: (i, 0)),
            pl.BlockSpec((_T, _G, _D), lambda i: (0, 0, 0)),
            pl.BlockSpec((_G, _T), lambda i: (0, 0)),
            pl.BlockSpec((_T, _D, _D), lambda i: (0, 0, 0)),
            pl.BlockSpec((_T, _D), lambda i: (0, 0)),
            pl.BlockSpec((_G, _D), lambda i: (0, 0)),
            pl.BlockSpec((_G, 1), lambda i: (0, 0)),
        ],
        out_specs=out_specs,
        out_shape=out_shape,
        scratch_shapes=[pltpu.VMEM((_T, _G, _D), _F32),
                        pltpu.VMEM((_G, _T), _F32)],
    )(agg, bat2, p, q, Ws, bs, SX, cnt)
    return res


def kernel(x, edge_index, edge_weight, batch, gamma, beta, Ws, bs):
    row = edge_index[0]
    col = edge_index[1]
    gamma2 = gamma.reshape(1, _D)
    beta2 = beta.reshape(1, _D)
    bat2 = batch.reshape(_N, 1)

    degp = _sc_col_degree(col, edge_weight)
    degp_t = degp.reshape(_NW, _NR * _D)[:, :_N].T
    stats = _tc_stats(x)
    xb, xs, dis = _tc_prep(x, stats, gamma2, beta2, degp_t)
    P = _sc_edge_agg(xs, row, col, edge_weight)
    agg, SX, cnt, p1, q1 = _tc_finalize(P[0], P[1], xs, xb, dis, bat2, Ws, bs)
    p12, q12 = _tc_node_pass(agg, bat2, p1, q1, Ws, bs, SX, cnt, final=False)
    (out,) = _tc_node_pass(agg, bat2, p12, q12, Ws, bs, SX, cnt, final=True)
    return out


# speedup vs baseline: 96.7782x; 1.0103x over previous
import functools

import jax
import jax.numpy as jnp
from jax import lax
from jax.experimental import pallas as pl
from jax.experimental.pallas import tpu as pltpu
from jax.experimental.pallas import tpu_sc as plsc

_N = 10000
_E = 320000
_D = 128
_T = 10
_G = 128

_NC = 2
_NS = 16
_NW = _NC * _NS


_NR = 80


def _sc_col_degree(col, ew):
    ept = _E // _NW
    ch = 2000
    nch = ept // ch
    mesh = plsc.VectorSubcoreMesh(core_axis_name="c", subcore_axis_name="s")

    @functools.partial(
        pl.kernel, mesh=mesh,
        out_type=jax.ShapeDtypeStruct((_NW, _NR, _D), jnp.float32),
        compiler_params=pltpu.CompilerParams(needs_layout_passes=False),
        scratch_types=[
            pltpu.VMEM((_NR, _D), jnp.float32),
            pltpu.VMEM((ch,), jnp.int32),
            pltpu.VMEM((ch,), jnp.float32),
        ],
    )
    def k(col_hbm, ew_hbm, out_hbm, acc, idxb, ewb):
        cc = lax.axis_index("c")
        ss = lax.axis_index("s")
        wid = ss * _NC + cc

        def zero(g, carry):
            acc[g // 8, pl.ds((g % 8) * 16, 16)] = jnp.zeros((16,), jnp.float32)
            return carry
        lax.fori_loop(0, _NR * (_D // 16), zero, 0)

        base = wid * ept

        def chunk(kk, carry):
            off = base + kk * ch
            pltpu.sync_copy(col_hbm.at[pl.ds(off, ch)], idxb)
            pltpu.sync_copy(ew_hbm.at[pl.ds(off, ch)], ewb)

            def grp(j, c2):
                idx = idxb[pl.ds(j * 16, 16)]
                w = ewb[pl.ds(j * 16, 16)]
                hi = lax.shift_right_logical(idx, 7)
                lo = lax.bitwise_and(idx, 127)
                plsc.addupdate_scatter(acc, [hi, lo], w)
                return c2
            lax.fori_loop(0, ch // 16, grp, 0)
            return carry
        lax.fori_loop(0, nch, chunk, 0)
        pltpu.sync_copy(acc, out_hbm.at[wid])

    return k(col, ew)


def _sc_edge_agg(xs, row, col, ew):
    epc = _E // _NC
    ept = epc // _NS
    ch = 40
    nch = ept // ch
    nbuf = 5
    rpt = 640
    mesh = plsc.VectorSubcoreMesh(core_axis_name="c", subcore_axis_name="s")

    scratch = [pltpu.VMEM_SHARED((_N, _D), jnp.float32)]
    for _ in range(nbuf):
        scratch += [
            pltpu.VMEM((ch,), jnp.int32),
            pltpu.VMEM((ch,), jnp.int32),
            pltpu.VMEM((ch,), jnp.float32),
            pltpu.VMEM((ch, _D), jnp.float32),
            pltpu.SemaphoreType.DMA,
            pltpu.SemaphoreType.DMA,
            pltpu.SemaphoreType.DMA,
        ]

    @functools.partial(
        pl.kernel, mesh=mesh,
        out_type=jax.ShapeDtypeStruct((_NC, _N, _D), jnp.float32),
        compiler_params=pltpu.CompilerParams(needs_layout_passes=False),
        scratch_types=scratch,
    )
    def k(xs_hbm, row_hbm, col_hbm, ew_hbm, out_hbm, shacc, *bufs):
        rowb = [bufs[7 * b + 0] for b in range(nbuf)]
        colb = [bufs[7 * b + 1] for b in range(nbuf)]
        ewb = [bufs[7 * b + 2] for b in range(nbuf)]
        rows = [bufs[7 * b + 3] for b in range(nbuf)]
        isem = [bufs[7 * b + 4] for b in range(nbuf)]
        gsem = [bufs[7 * b + 5] for b in range(nbuf)]
        ssem = [bufs[7 * b + 6] for b in range(nbuf)]
        cc = lax.axis_index("c")
        ss = lax.axis_index("s")

        def zzb(g, carry):
            rows[0][g // 8, pl.ds((g % 8) * 16, 16)] = (
                jnp.zeros((16,), jnp.float32))
            return carry
        lax.fori_loop(0, ch * (_D // 16), zzb, 0)

        r0 = ss * rpt
        nz = lax.select(ss == _NS - 1, (_N - 15 * rpt) // ch, rpt // ch)

        def zsh(i, carry):
            pltpu.sync_copy(rows[0], shacc.at[pl.ds(r0 + i * ch, ch)])
            return carry
        lax.fori_loop(0, nz, zsh, 0)
        plsc.subcore_barrier()

        base = cc * epc + ss * ept

        def fetch_idx(cp, bb):
            off = base + cp * ch
            pltpu.async_copy(row_hbm.at[pl.ds(off, ch)], rowb[bb], isem[bb])
            pltpu.async_copy(col_hbm.at[pl.ds(off, ch)], colb[bb], isem[bb])
            pltpu.async_copy(ew_hbm.at[pl.ds(off, ch)], ewb[bb], isem[bb])

        def wait_idx(cp, bb):
            off = base + cp * ch
            pltpu.make_async_copy(
                row_hbm.at[pl.ds(off, ch)], rowb[bb], isem[bb]).wait()
            pltpu.make_async_copy(
                col_hbm.at[pl.ds(off, ch)], colb[bb], isem[bb]).wait()
            pltpu.make_async_copy(
                ew_hbm.at[pl.ds(off, ch)], ewb[bb], isem[bb]).wait()

        for b in range(3):
            fetch_idx(b, b)
        for b in range(2):
            wait_idx(b, b)
            pltpu.async_copy(xs_hbm.at[rowb[b]], rows[b], gsem[b])

        def outer(kk, carry):
            for b in range(nbuf):
                c = kk * nbuf + b
                b3 = (b + 3) % nbuf
                b2 = (b + 2) % nbuf

                @pl.when(c + 3 < nch)
                def _(b3=b3, c=c):
                    @pl.when(c + 3 >= nbuf)
                    def _():
                        pltpu.make_async_copy(
                            rows[b3], shacc.at[colb[b3]], ssem[b3]).wait()
                    fetch_idx(c + 3, b3)

                @pl.when(c + 2 < nch)
                def _(b2=b2, c=c):
                    wait_idx(c + 2, b2)
                    pltpu.async_copy(xs_hbm.at[rowb[b2]], rows[b2], gsem[b2])

                pltpu.make_async_copy(xs_hbm.at[rowb[b]], rows[b],
                                      gsem[b]).wait()

                def sc8(i8, c2, b=b):
                    for u in range(8):
                        j = i8 * 8 + u
                        wj = plsc.load_gather(
                            ewb[b], [jnp.full((16,), j, jnp.int32)])
                        for v in range(_D // 16):
                            sl = pl.ds(v * 16, 16)
                            rows[b][j, sl] = rows[b][j, sl] * wj
                    return c2
                lax.fori_loop(0, ch // 8, sc8, 0)
                pltpu.async_copy(rows[b], shacc.at[colb[b]], ssem[b],
                                 add=True)
            return carry
        lax.fori_loop(0, nch // nbuf, outer, 0)
        for b in range(nbuf):
            pltpu.make_async_copy(rows[b], shacc.at[colb[b]], ssem[b]).wait()
        plsc.subcore_barrier()

        def wout(i, carry):
            pltpu.sync_copy(shacc.at[pl.ds(r0 + i * ch, ch)],
                            out_hbm.at[cc, pl.ds(r0 + i * ch, ch)])
            return carry
        lax.fori_loop(0, nz, wout, 0)

    return k(xs, row, col, ew)


_BN = 2000
_NB = _N // _BN

_F32 = jnp.float32
_DN0 = (((0,), (0,)), ((), ()))
_DN1 = (((1,), (1,)), ((), ()))


def _onehot(bat):
    return (bat == lax.broadcasted_iota(jnp.int32, (1, _G), 1)).astype(_F32)


def _squash_rows(s):
    n2 = jnp.sum(s * s, axis=1, keepdims=True)
    return (n2 / (1.0 + n2)) * s / jnp.sqrt(n2 + 1e-8)


def _tc_prep(x, gamma2, beta2, degp_t, bat2):
    def body(x_ref, g_ref, b_ref, dp_ref, bat_ref,
             xs_ref, dis_ref, SX_ref, cnt_ref, st_sc):
        p = pl.program_id(0)
        i = pl.program_id(1)
        xv = x_ref[...]

        @pl.when(p == 0)
        def _():
            blk = jnp.concatenate(
                [jnp.sum(xv, axis=0, keepdims=True),
                 jnp.sum(xv * xv, axis=0, keepdims=True)], axis=0)

            @pl.when(i == 0)
            def _():
                st_sc[...] = blk

            @pl.when(i > 0)
            def _():
                st_sc[...] += blk

        @pl.when(p == 1)
        def _():
            mean = st_sc[0:1, :] * (1.0 / _N)
            var = st_sc[1:2, :] * (1.0 / _N) - mean * mean
            xb = (xv - mean) * lax.rsqrt(var + 1e-5) * g_ref[...] + b_ref[...]
            deg = jnp.sum(dp_ref[...], axis=1, keepdims=True) + 1.0
            dis = lax.rsqrt(deg)
            xs_ref[...] = xb * dis
            dis_ref[...] = dis
            M = _onehot(bat_ref[...])
            sx = lax.dot_general(M, xb, _DN0, preferred_element_type=_F32)
            cn = lax.dot_general(M, jnp.ones((_BN, 1), _F32), _DN0,
                                 preferred_element_type=_F32)

            @pl.when(i == 0)
            def _():
                SX_ref[...] = sx
                cnt_ref[...] = cn

            @pl.when(i > 0)
            def _():
                SX_ref[...] += sx
                cnt_ref[...] += cn

    return pl.pallas_call(
        body,
        grid=(2, _NB),
        in_specs=[
            pl.BlockSpec((_BN, _D), lambda p, i: (i, 0)),
            pl.BlockSpec((1, _D), lambda p, i: (0, 0)),
            pl.BlockSpec((1, _D), lambda p, i: (0, 0)),
            pl.BlockSpec((_BN, _NW), lambda p, i: (i, 0)),
            pl.BlockSpec((_BN, 1), lambda p, i: (i, 0)),
        ],
        out_specs=[
            pl.BlockSpec((_BN, _D), lambda p, i: (i, 0)),
            pl.BlockSpec((_BN, 1), lambda p, i: (i, 0)),
            pl.BlockSpec((_G, _D), lambda p, i: (0, 0)),
            pl.BlockSpec((_G, 1), lambda p, i: (0, 0)),
        ],
        out_shape=[
            jax.ShapeDtypeStruct((_N, _D), _F32),
            jax.ShapeDtypeStruct((_N, 1), _F32),
            jax.ShapeDtypeStruct((_G, _D), _F32),
            jax.ShapeDtypeStruct((_G, 1), _F32),
        ],
        scratch_shapes=[pltpu.VMEM((2, _D), _F32)],
    )(x, gamma2, beta2, degp_t, bat2)


def _tc_route(P, xs, dis, bat2, Ws, bs, SX, cnt):
    def body(P_ref, xs_ref, dis_ref, bat_ref, Ws_ref, bs_ref, SX_ref, cnt_ref,
             out_ref, SA_sc, p_sc, q_sc, A_sc, m_sc):
        p = pl.program_id(0)
        i = pl.program_id(1)
        agg = dis_ref[...] * (P_ref[0] + P_ref[1] + xs_ref[...])
        M = _onehot(bat_ref[...])

        @pl.when(p == 0)
        def _():
            sa = lax.dot_general(M, agg, _DN0, preferred_element_type=_F32)

            @pl.when(i == 0)
            def _():
                SA_sc[...] = sa

            @pl.when(i > 0)
            def _():
                SA_sc[...] += sa

            @pl.when(i == _NB - 1)
            def _():
                SA = SA_sc[...]
                cnt = cnt_ref[...]
                for t in range(_T):
                    Wt = Ws_ref[t]
                    bt = bs_ref[t:t + 1, :]
                    s1 = (jnp.dot(SA, Wt, preferred_element_type=_F32)
                          + cnt * bt) * (1.0 / _T)
                    v1 = _squash_rows(s1)
                    p_sc[t] = lax.dot_general(v1, Wt, _DN1,
                                              preferred_element_type=_F32)
                    q_sc[:, t:t + 1] = lax.dot_general(
                        v1, bt, _DN1, preferred_element_type=_F32)

        @pl.when(p > 0)
        def _():
            qn = jnp.dot(M, q_sc[...], preferred_element_type=_F32)
            wcols = []
            for t in range(_T):
                pn = jnp.dot(M, p_sc[t], preferred_element_type=_F32)
                wcols.append(jnp.sum(agg * pn, axis=1, keepdims=True))
            logits = jnp.concatenate(wcols, axis=1) + qn
            mx = jnp.max(logits, axis=1, keepdims=True)
            e = jnp.exp(logits - mx)
            c = e / jnp.sum(e, axis=1, keepdims=True)

            @pl.when(i == 0)
            def _():
                A_sc[...] = jnp.zeros_like(A_sc)
                m_sc[...] = jnp.zeros_like(m_sc)

            m_sc[...] += lax.dot_general(M, c, _DN0,
                                         preferred_element_type=_F32)
            for t in range(_T):
                A_sc[t] += lax.dot_general(M, c[:, t:t + 1] * agg, _DN0,
                                           preferred_element_type=_F32)

            @pl.when(i == _NB - 1)
            def _():

                @pl.when(p == 1)
                def _():
                    for t in range(_T):
                        Wt = Ws_ref[t]
                        bt = bs_ref[t:t + 1, :]
                        s = (jnp.dot(A_sc[t], Wt, preferred_element_type=_F32)
                             + m_sc[:, t:t + 1] * bt)
                        v = _squash_rows(s)
                        p_sc[t] += lax.dot_general(
                            v, Wt, _DN1, preferred_element_type=_F32)
                        q_sc[:, t:t + 1] += lax.dot_general(
                            v, bt, _DN1, preferred_element_type=_F32)

                @pl.when(p == 2)
                def _():
                    xmean = SX_ref[...] / jnp.maximum(cnt_ref[...], 1.0)
                    for t in range(_T):
                        Wt = Ws_ref[t]
                        bt = bs_ref[t:t + 1, :]
                        s = (jnp.dot(A_sc[t], Wt, preferred_element_type=_F32)
                             + m_sc[:, t:t + 1] * bt) + xmean
                        v = _squash_rows(s)
                        out_ref[:, t:t + 1] = jnp.sqrt(
                            jnp.sum(v * v, axis=1, keepdims=True))

    return pl.pallas_call(
        body,
        grid=(3, _NB),
        in_specs=[
            pl.BlockSpec((2, _BN, _D), lambda p, i: (0, i, 0)),
            pl.BlockSpec((_BN, _D), lambda p, i: (i, 0)),
            pl.BlockSpec((_BN, 1), lambda p, i: (i, 0)),
            pl.BlockSpec((_BN, 1), lambda p, i: (i, 0)),
            pl.BlockSpec((_T, _D, _D), lambda p, i: (0, 0, 0)),
            pl.BlockSpec((_T, _D), lambda p, i: (0, 0)),
            pl.BlockSpec((_G, _D), lambda p, i: (0, 0)),
            pl.BlockSpec((_G, 1), lambda p, i: (0, 0)),
        ],
        out_specs=pl.BlockSpec((_G, _T), lambda p, i: (0, 0)),
        out_shape=jax.ShapeDtypeStruct((_G, _T), _F32),
        scratch_shapes=[
            pltpu.VMEM((_G, _D), _F32),
            pltpu.VMEM((_T, _G, _D), _F32),
            pltpu.VMEM((_G, _T), _F32),
            pltpu.VMEM((_T, _G, _D), _F32),
            pltpu.VMEM((_G, _T), _F32),
        ],
    )(P, xs, dis, bat2, Ws, bs, SX, cnt)


def kernel(x, edge_index, edge_weight, batch, gamma, beta, Ws, bs):
    row = edge_index[0]
    col = edge_index[1]
    gamma2 = gamma.reshape(1, _D)
    beta2 = beta.reshape(1, _D)
    bat2 = batch.reshape(_N, 1)

    degp = _sc_col_degree(col, edge_weight)
    degp_t = degp.reshape(_NW, _NR * _D)[:, :_N].T
    xs, dis, SX, cnt = _tc_prep(x, gamma2, beta2, degp_t, bat2)
    P = _sc_edge_agg(xs, row, col, edge_weight)
    return _tc_route(P, xs, dis, bat2, Ws, bs, SX, cnt)
